# baseline (speedup 1.0000x reference)
import functools

import jax
import jax.numpy as jnp
from jax import lax
from jax.experimental import pallas as pl
from jax.experimental.pallas import tpu as pltpu
from jax.experimental.pallas import tpu_sc as plsc


def _proj_body(tableT_ref, wt_ref, b_ref, s_ref):
    j = pl.program_id(0)
    partial = jnp.sum(tableT_ref[0] * wt_ref[0], axis=0)

    @pl.when(j == 0)
    def _init():
        s_ref[...] = partial + b_ref[0]

    @pl.when(j > 0)
    def _acc():
        s_ref[...] = s_ref[...] + partial


def _project_table(tableT, Wt, b, block_rows=12):
    D, V = tableT.shape
    grid = D // block_rows
    t3 = tableT.reshape(grid, block_rows, V)
    w3 = Wt.reshape(grid, block_rows, 1)
    return pl.pallas_call(
        _proj_body,
        grid=(grid,),
        in_specs=[
            pl.BlockSpec((1, block_rows, V), lambda j: (j, 0, 0)),
            pl.BlockSpec((1, block_rows, 1), lambda j: (j, 0, 0)),
            pl.BlockSpec(memory_space=pltpu.SMEM),
        ],
        out_specs=pl.BlockSpec((V,), lambda j: (0,)),
        out_shape=jax.ShapeDtypeStruct((V,), jnp.float32),
    )(t3, w3, b)


def _make_pool_kernel(L, B, V, n_workers, lanes):
    bw = B // n_workers
    chunks = bw // lanes
    mesh = plsc.VectorSubcoreMesh(core_axis_name="c", subcore_axis_name="s")
    nc = 2

    @functools.partial(
        pl.kernel,
        out_type=jax.ShapeDtypeStruct((B,), jnp.float32),
        mesh=mesh,
        scratch_types=[
            pltpu.VMEM((L, bw), jnp.int32),
            pltpu.VMEM((L, bw), jnp.float32),
            pltpu.VMEM((bw,), jnp.float32),
            pltpu.SemaphoreType.DMA,
        ],
    )
    def pool(s_hbm, text_hbm, out_hbm, idx_v, gat_v, res_v, sem):
        wid = lax.axis_index("s") * nc + lax.axis_index("c")
        b0 = wid * bw
        pltpu.sync_copy(text_hbm.at[:, pl.ds(b0, bw)], idx_v)
        cps = [
            pltpu.async_copy(s_hbm.at[idx_v.at[l]], gat_v.at[l], sem)
            for l in range(L)
        ]
        for cp in cps:
            cp.wait()
        inv_l = jnp.float32(1.0 / L)
        for j in range(chunks):
            acc = jnp.zeros((lanes,), jnp.float32)
            for l in range(L):
                acc = acc + gat_v[l, pl.ds(j * lanes, lanes)]
            res_v[pl.ds(j * lanes, lanes)] = acc * inv_l
        pltpu.sync_copy(res_v, out_hbm.at[pl.ds(b0, bw)])

    return pool


def kernel(text, table, W, b):
    L, B = text.shape
    V = table.shape[0]
    s = _project_table(table.T, W.T, b)
    pool = _make_pool_kernel(L, B, V, n_workers=32, lanes=16)
    out = pool(s, text)
    return out.reshape(B, 1)


# speedup vs baseline: 2.2684x; 2.2684x over previous
import functools

import jax
import jax.numpy as jnp
from jax import lax
from jax.experimental import pallas as pl
from jax.experimental.pallas import tpu as pltpu
from jax.experimental.pallas import tpu_sc as plsc

_SC_COLS = 24576


def _proj_body(tableT_ref, wt_ref, s_ref):
    s_ref[...] = jnp.sum(tableT_ref[...] * wt_ref[...], axis=0)


def _project_table_tc(tableT, Wt, v0, block_cols=8192):
    D, V = tableT.shape
    v_tc = V - v0
    skip = v0 // block_cols
    grid = (v_tc + block_cols - 1) // block_cols
    return pl.pallas_call(
        _proj_body,
        grid=(grid,),
        in_specs=[
            pl.BlockSpec((D, block_cols), lambda i: (0, i + skip)),
            pl.BlockSpec((D, 1), lambda i: (0, 0)),
        ],
        out_specs=pl.BlockSpec((block_cols,), lambda i: (i,)),
        out_shape=jax.ShapeDtypeStruct((v_tc,), jnp.float32),
    )(tableT, Wt)


def _make_sc_proj(D, v0, vsc, n_workers=32, lanes=16, chunk=256):
    cw = vsc // n_workers
    nchunks = cw // chunk
    groups = chunk // lanes
    mesh = plsc.VectorSubcoreMesh(core_axis_name="c", subcore_axis_name="s")
    nc = 2

    d_blocks = D // lanes
    d_tail = D - d_blocks * lanes

    @functools.partial(
        pl.kernel,
        out_type=jax.ShapeDtypeStruct((vsc,), jnp.float32),
        mesh=mesh,
        scratch_types=[
            pltpu.VMEM((D, chunk), jnp.float32),
            pltpu.VMEM((D,), jnp.float32),
            pltpu.VMEM((chunk,), jnp.float32),
        ],
    )
    def sc_proj(tT_hbm, w_hbm, out_hbm, buf_v, w_v, res_v):
        wid = lax.axis_index("s") * nc + lax.axis_index("c")
        base = wid * cw
        pltpu.sync_copy(w_hbm, w_v)
        for k in range(nchunks):
            pltpu.sync_copy(
                tT_hbm.at[:, pl.ds(v0 + base + k * chunk, chunk)], buf_v
            )

            def body(bi, accs):
                d0 = bi * lanes
                wvec = w_v[pl.ds(d0, lanes)]
                for i in range(lanes):
                    w = wvec[i]
                    accs = tuple(
                        accs[g] + buf_v[d0 + i, pl.ds(g * lanes, lanes)] * w
                        for g in range(groups)
                    )
                return accs

            accs = lax.fori_loop(
                0, d_blocks, body,
                tuple(jnp.zeros((lanes,), jnp.float32) for _ in range(groups)),
            )
            if d_tail:
                wvec = w_v[pl.ds(D - lanes, lanes)]
                for i in range(lanes - d_tail, lanes):
                    w = wvec[i]
                    d = D - lanes + i
                    accs = tuple(
                        accs[g] + buf_v[d, pl.ds(g * lanes, lanes)] * w
                        for g in range(groups)
                    )
            for g in range(groups):
                res_v[pl.ds(g * lanes, lanes)] = accs[g]
            pltpu.sync_copy(res_v, out_hbm.at[pl.ds(base + k * chunk, chunk)])

    return sc_proj


def _make_pool_kernel(L, B, V, n_workers, lanes):
    bw = B // n_workers
    chunks = bw // lanes
    mesh = plsc.VectorSubcoreMesh(core_axis_name="c", subcore_axis_name="s")
    nc = 2

    @functools.partial(
        pl.kernel,
        out_type=jax.ShapeDtypeStruct((B,), jnp.float32),
        mesh=mesh,
        scratch_types=[
            pltpu.VMEM((L, bw), jnp.int32),
            pltpu.VMEM((L, bw), jnp.float32),
            pltpu.VMEM((bw,), jnp.float32),
            pltpu.SemaphoreType.DMA,
        ],
    )
    def pool(s_hbm, text_hbm, out_hbm, idx_v, gat_v, res_v, sem):
        wid = lax.axis_index("s") * nc + lax.axis_index("c")
        b0 = wid * bw
        pltpu.sync_copy(text_hbm.at[:, pl.ds(b0, bw)], idx_v)
        cps = [
            pltpu.async_copy(s_hbm.at[idx_v.at[l]], gat_v.at[l], sem)
            for l in range(L)
        ]
        for cp in cps:
            cp.wait()
        inv_l = jnp.float32(1.0 / L)
        for j in range(chunks):
            acc = jnp.zeros((lanes,), jnp.float32)
            for l in range(L):
                acc = acc + gat_v[l, pl.ds(j * lanes, lanes)]
            res_v[pl.ds(j * lanes, lanes)] = acc * inv_l
        pltpu.sync_copy(res_v, out_hbm.at[pl.ds(b0, bw)])

    return pool


def kernel(text, table, W, b):
    L, B = text.shape
    V, D = table.shape
    tableT = table.T
    s_tc = _project_table_tc(tableT, W.T, _SC_COLS)
    sc_proj = _make_sc_proj(D, 0, _SC_COLS)
    s_sc = sc_proj(tableT, W.reshape(D))
    s = jnp.concatenate([s_sc, s_tc])
    pool = _make_pool_kernel(L, B, V, n_workers=32, lanes=16)
    out = pool(s, text)
    return out.reshape(B, 1) + b


# speedup vs baseline: 2.9685x; 1.3086x over previous
import functools

import jax
import jax.numpy as jnp
from jax import lax
from jax.experimental import pallas as pl
from jax.experimental.pallas import tpu as pltpu
from jax.experimental.pallas import tpu_sc as plsc


def _proj_body(tableT_ref, wt_ref, b_ref, s_ref):
    s_ref[...] = jnp.sum(tableT_ref[...] * wt_ref[...], axis=0) + b_ref[0]


def _project_table(tableT, Wt, b, block_cols=8192):
    D, V = tableT.shape
    grid = (V + block_cols - 1) // block_cols
    return pl.pallas_call(
        _proj_body,
        grid=(grid,),
        in_specs=[
            pl.BlockSpec((D, block_cols), lambda i: (0, i)),
            pl.BlockSpec((D, 1), lambda i: (0, 0)),
            pl.BlockSpec(memory_space=pltpu.SMEM),
        ],
        out_specs=pl.BlockSpec((block_cols,), lambda i: (i,)),
        out_shape=jax.ShapeDtypeStruct((V,), jnp.float32),
    )(tableT, Wt, b)


def _make_pool_kernel(L, B, V, n_workers, lanes):
    bw = B // n_workers
    chunks = bw // lanes
    mesh = plsc.VectorSubcoreMesh(core_axis_name="c", subcore_axis_name="s")
    nc = 2

    @functools.partial(
        pl.kernel,
        out_type=jax.ShapeDtypeStruct((B,), jnp.float32),
        mesh=mesh,
        scratch_types=[
            pltpu.VMEM((L, bw), jnp.int32),
            pltpu.VMEM((L, bw), jnp.float32),
            pltpu.VMEM((bw,), jnp.float32),
            pltpu.SemaphoreType.DMA,
        ],
    )
    def pool(s_hbm, text_hbm, out_hbm, idx_v, gat_v, res_v, sem):
        wid = lax.axis_index("s") * nc + lax.axis_index("c")
        b0 = wid * bw
        pltpu.sync_copy(text_hbm.at[:, pl.ds(b0, bw)], idx_v)
        cps = [
            pltpu.async_copy(s_hbm.at[idx_v.at[l]], gat_v.at[l], sem)
            for l in range(L)
        ]
        for cp in cps:
            cp.wait()
        inv_l = jnp.float32(1.0 / L)
        for j in range(chunks):
            acc = jnp.zeros((lanes,), jnp.float32)
            for l in range(L):
                acc = acc + gat_v[l, pl.ds(j * lanes, lanes)]
            res_v[pl.ds(j * lanes, lanes)] = acc * inv_l
        pltpu.sync_copy(res_v, out_hbm.at[pl.ds(b0, bw)])

    return pool


def kernel(text, table, W, b):
    L, B = text.shape
    V = table.shape[0]
    s = _project_table(table.T, W.T, b)
    pool = _make_pool_kernel(L, B, V, n_workers=32, lanes=16)
    out = pool(s, text)
    return out.reshape(B, 1)
